# Initial kernel scaffold; baseline (speedup 1.0000x reference)
import jax
import jax.numpy as jnp
from jax.experimental import pallas as pl


def kernel(x, edge_index, relations, Wl1, bl1, Wr1, Wl2, bl2, Wr2, Wl3, bl3, Wr3):
    raise NotImplementedError("write your pallas kernel here")


# speedup vs baseline: 6.7631x; 6.7631x over previous
import functools
import jax
import jax.numpy as jnp
from jax import lax
from jax.experimental import pallas as pl
from jax.experimental.pallas import tpu as pltpu
from jax.experimental.pallas import tpu_sc as plsc

NC, NS, LANES = 2, 16, 16
CHUNK = 128


@functools.lru_cache(maxsize=None)
def _make_segsum(npad, e, w):
    n_chunks = e // CHUNK
    nw = NC * NS
    rows_per_tile = npad // NS
    wb_chunks = rows_per_tile // CHUNK
    mesh = plsc.VectorSubcoreMesh(core_axis_name="c", subcore_axis_name="s")

    def body(y_hbm, ei_hbm, p_out, ei_v, rows_v, acc, sem):
        ci = lax.axis_index("c")
        si = lax.axis_index("s")
        wid = si * NC + ci
        zeros16 = jnp.zeros((LANES,), jnp.float32)

        @pl.loop(0, CHUNK)
        def _zr(r):
            for j in range(w // LANES):
                rows_v[r, pl.ds(j * LANES, LANES)] = zeros16

        row0 = si * rows_per_tile

        @pl.loop(0, wb_chunks)
        def _za(k):
            pltpu.sync_copy(rows_v, acc.at[pl.ds(row0 + k * CHUNK, CHUNK)])

        plsc.subcore_barrier()

        lo = wid * n_chunks // nw
        hi = (wid + 1) * n_chunks // nw

        @pl.loop(lo, hi)
        def _edges(c):
            pltpu.sync_copy(ei_hbm.at[c], ei_v)
            pltpu.async_copy(y_hbm.at[ei_v.at[0]], rows_v, sem).wait()
            pltpu.sync_copy(rows_v, acc.at[ei_v.at[1]], add=True)

        plsc.subcore_barrier()

        @pl.loop(0, wb_chunks)
        def _wb(k):
            r0 = row0 + k * CHUNK
            pltpu.sync_copy(acc.at[pl.ds(r0, CHUNK)], rows_v)
            pltpu.sync_copy(rows_v, p_out.at[ci, pl.ds(r0, CHUNK)])

    return pl.kernel(
        body,
        out_type=jax.ShapeDtypeStruct((NC, npad, w), jnp.float32),
        mesh=mesh,
        scratch_types=(
            pltpu.VMEM((2, CHUNK), jnp.int32),
            pltpu.VMEM((CHUNK, w), jnp.float32),
            pltpu.VMEM_SHARED((npad, w), jnp.float32),
            pltpu.SemaphoreType.DMA,
        ),
    )


@functools.lru_cache(maxsize=None)
def _make_deg(npad, e):
    w = 128
    n_chunks = e // CHUNK
    nw = NC * NS
    rows_per_tile = npad // NS
    wb_chunks = rows_per_tile // CHUNK
    mesh = plsc.VectorSubcoreMesh(core_axis_name="c", subcore_axis_name="s")

    def body(ei_hbm, d_out, ei_v, rows_v, dacc, sem):
        ci = lax.axis_index("c")
        si = lax.axis_index("s")
        wid = si * NC + ci
        zeros16 = jnp.zeros((LANES,), jnp.float32)

        @pl.loop(0, CHUNK)
        def _zr(r):
            for j in range(w // LANES):
                rows_v[r, pl.ds(j * LANES, LANES)] = zeros16

        row0 = si * rows_per_tile

        @pl.loop(0, wb_chunks)
        def _za(k):
            pltpu.sync_copy(rows_v, dacc.at[pl.ds(row0 + k * CHUNK, CHUNK)])

        ones16 = jnp.full((LANES,), 1.0, jnp.float32)

        @pl.loop(0, CHUNK)
        def _fo(r):
            rows_v[r, pl.ds(0, LANES)] = ones16

        plsc.subcore_barrier()

        lo = wid * n_chunks // nw
        hi = (wid + 1) * n_chunks // nw

        @pl.loop(lo, hi)
        def _edges(c):
            pltpu.sync_copy(ei_hbm.at[c], ei_v)
            pltpu.sync_copy(rows_v, dacc.at[ei_v.at[1]], add=True)

        plsc.subcore_barrier()

        @pl.loop(0, wb_chunks)
        def _wb(k):
            r0 = row0 + k * CHUNK
            pltpu.sync_copy(dacc.at[pl.ds(r0, CHUNK)], rows_v)
            pltpu.sync_copy(rows_v, d_out.at[ci, pl.ds(r0, CHUNK)])

    return pl.kernel(
        body,
        out_type=jax.ShapeDtypeStruct((NC, npad, w), jnp.float32),
        mesh=mesh,
        scratch_types=(
            pltpu.VMEM((2, CHUNK), jnp.int32),
            pltpu.VMEM((CHUNK, w), jnp.float32),
            pltpu.VMEM_SHARED((npad, w), jnp.float32),
            pltpu.SemaphoreType.DMA,
        ),
    )


def _mm_first(npad, d, h, r):
    def body(x_ref, wl_ref, wr_ref, y_ref, z_ref):
        xb = x_ref[...]
        y_ref[...] = jnp.dot(xb, wl_ref[...], preferred_element_type=jnp.float32)
        z_ref[...] = jnp.dot(xb, wr_ref[...], preferred_element_type=jnp.float32)

    return pl.pallas_call(
        body,
        grid=(npad // r,),
        in_specs=[
            pl.BlockSpec((r, d), lambda i: (i, 0)),
            pl.BlockSpec((d, h), lambda i: (0, 0)),
            pl.BlockSpec((d, h), lambda i: (0, 0)),
        ],
        out_specs=[
            pl.BlockSpec((r, h), lambda i: (i, 0)),
            pl.BlockSpec((r, h), lambda i: (i, 0)),
        ],
        out_shape=[
            jax.ShapeDtypeStruct((npad, h), jnp.float32),
            jax.ShapeDtypeStruct((npad, h), jnp.float32),
        ],
    )


def _mm_combine(npad, w_in, w_out, r):
    def body(p_ref, d_ref, z_ref, b_ref, wl_ref, wr_ref, y_ref, z2_ref):
        s = p_ref[0] + p_ref[1]
        deg = d_ref[0, :, 0] + d_ref[1, :, 0]
        dc = jnp.maximum(deg, 1.0)
        hh = s / dc[:, None] + b_ref[...] + z_ref[...]
        hh = jnp.maximum(hh, 0.0)
        y_ref[...] = jnp.dot(hh, wl_ref[...], preferred_element_type=jnp.float32)
        z2_ref[...] = jnp.dot(hh, wr_ref[...], preferred_element_type=jnp.float32)

    return pl.pallas_call(
        body,
        grid=(npad // r,),
        in_specs=[
            pl.BlockSpec((NC, r, w_in), lambda i: (0, i, 0)),
            pl.BlockSpec((NC, r, 128), lambda i: (0, i, 0)),
            pl.BlockSpec((r, w_in), lambda i: (i, 0)),
            pl.BlockSpec((1, w_in), lambda i: (0, 0)),
            pl.BlockSpec((w_in, w_out), lambda i: (0, 0)),
            pl.BlockSpec((w_in, w_out), lambda i: (0, 0)),
        ],
        out_specs=[
            pl.BlockSpec((r, w_out), lambda i: (i, 0)),
            pl.BlockSpec((r, w_out), lambda i: (i, 0)),
        ],
        out_shape=[
            jax.ShapeDtypeStruct((npad, w_out), jnp.float32),
            jax.ShapeDtypeStruct((npad, w_out), jnp.float32),
        ],
    )


def _mm_final(npad, w, r):
    def body(p_ref, d_ref, z_ref, b_ref, o_ref):
        s = p_ref[0] + p_ref[1]
        deg = d_ref[0, :, 0] + d_ref[1, :, 0]
        dc = jnp.maximum(deg, 1.0)
        o_ref[...] = s / dc[:, None] + b_ref[...] + z_ref[...]

    return pl.pallas_call(
        body,
        grid=(npad // r,),
        in_specs=[
            pl.BlockSpec((NC, r, w), lambda i: (0, i, 0)),
            pl.BlockSpec((NC, r, 128), lambda i: (0, i, 0)),
            pl.BlockSpec((r, w), lambda i: (i, 0)),
            pl.BlockSpec((1, w), lambda i: (0, 0)),
        ],
        out_specs=pl.BlockSpec((r, w), lambda i: (i, 0)),
        out_shape=jax.ShapeDtypeStruct((npad, w), jnp.float32),
    )


def kernel(x, edge_index, relations, Wl1, bl1, Wr1, Wl2, bl2, Wr2, Wl3, bl3, Wr3):
    n, d = x.shape
    e = edge_index.shape[1]
    h = Wl1.shape[1]
    c = Wl3.shape[1]
    cpad = 128
    tile = NS * CHUNK
    npad = ((n + tile - 1) // tile) * tile
    r = 1024

    x_pad = jnp.pad(x, ((0, npad - n), (0, 0)))
    ei = edge_index.astype(jnp.int32).reshape(2, e // CHUNK, CHUNK)
    ei = jnp.transpose(ei, (1, 0, 2))
    wl3p = jnp.pad(Wl3, ((0, 0), (0, cpad - c)))
    wr3p = jnp.pad(Wr3, ((0, 0), (0, cpad - c)))
    bl3p = jnp.pad(bl3, (0, cpad - c)).reshape(1, cpad)

    dpart = _make_deg(npad, e)(ei)
    y1, z1 = _mm_first(npad, d, h, r)(x_pad, Wl1, Wr1)
    p1 = _make_segsum(npad, e, h)(y1, ei)
    y2, z2 = _mm_combine(npad, h, h, r)(p1, dpart, z1, bl1.reshape(1, h), Wl2, Wr2)
    p2 = _make_segsum(npad, e, h)(y2, ei)
    y3, z3 = _mm_combine(npad, h, cpad, r)(p2, dpart, z2, bl2.reshape(1, h), wl3p, wr3p)
    p3 = _make_segsum(npad, e, cpad)(y3, ei)
    out = _mm_final(npad, cpad, r)(p3, dpart, z3, bl3p)
    return out[:n, :c]
